# baseline (speedup 1.0000x reference)
import functools

import jax
import jax.numpy as jnp
from jax.experimental import pallas as pl
from jax.experimental.pallas import tpu as pltpu

_NS = [1024, 2048, 1536, 512]
_H = 256
_HC = 3 * _H


def _lin_body(x_ref, wl_ref, bl_ref, wc_ref, bc_ref, o_ref):
    x = jnp.dot(x_ref[...], wl_ref[...], preferred_element_type=jnp.float32)
    x = x + bl_ref[...]
    o_ref[...] = jnp.dot(x, wc_ref[...], preferred_element_type=jnp.float32) + bc_ref[...]


def _lin_stage(emb, w_lin, b_lin2, wc, bc, bm=512):
    n, c = emb.shape
    return pl.pallas_call(
        _lin_body,
        grid=(n // bm,),
        in_specs=[
            pl.BlockSpec((bm, c), lambda i: (i, 0)),
            pl.BlockSpec((c, _H), lambda i: (0, 0)),
            pl.BlockSpec((1, _H), lambda i: (0, 0)),
            pl.BlockSpec((_H, _HC), lambda i: (0, 0)),
            pl.BlockSpec((1, _HC), lambda i: (0, 0)),
        ],
        out_specs=pl.BlockSpec((bm, _HC), lambda i: (i, 0)),
        out_shape=jax.ShapeDtypeStruct((n, _HC), jnp.float32),
    )(emb, w_lin, b_lin2, wc, bc)


def _attn_body(has_low, has_up, has_next, emit_mask, bm, *refs):
    it = iter(refs)
    h_ref = next(it)
    a_ref = next(it)
    lap_ref = next(it)
    if has_low:
        bl_ref = next(it)
        ylow_ref = next(it)
    if has_up:
        bu_ref = next(it)
        yup_ref = next(it)
    if has_next:
        wn_ref = next(it)
        bn_ref = next(it)
    o_ref = next(it)
    if emit_mask:
        m_ref = next(it)
    sd_ref = next(it)

    i = pl.program_id(0)
    h = h_ref[...]
    hb = h_ref[pl.ds(i * bm, bm), :]
    a = a_ref[...]

    acc = None
    if has_low:
        acc = jax.lax.dot_general(
            bl_ref[...], ylow_ref[...],
            dimension_numbers=(((0,), (0,)), ((), ())),
            preferred_element_type=jnp.float32,
        )
    if has_up:
        up = jnp.dot(bu_ref[...], yup_ref[...], preferred_element_type=jnp.float32)
        acc = up if acc is None else acc + up

    @pl.when(i == 0)
    def _():
        sd_ref[...] = jnp.sum(h * a[1:2, :], axis=1)[None, :]

    s_src = jnp.sum(hb * a[0:1, :], axis=1, keepdims=True)
    e = s_src + sd_ref[...]
    e = jnp.maximum(e, 0.2 * e)
    nz = lap_ref[...] != 0
    if emit_mask:
        m_ref[...] = nz.astype(jnp.int8)
    e = jnp.where(nz, e, -1e9)
    m = jnp.max(e, axis=1, keepdims=True)
    p = jnp.exp(e - m)
    out = jnp.dot(p, h, preferred_element_type=jnp.float32)
    out = out / jnp.sum(p, axis=1, keepdims=True)
    if acc is not None:
        out = out + acc
    x = jnp.maximum(out, 0.0)
    if has_next:
        o_ref[...] = jnp.dot(x, wn_ref[...], preferred_element_type=jnp.float32) + bn_ref[...]
    else:
        o_ref[...] = x


def _attn_stage(hcat, a2, lap, bnd_low, hcat_low, bnd_up, hcat_up, wn, bn, bm, emit_mask):
    n = hcat.shape[0]
    has_low = bnd_low is not None
    has_up = bnd_up is not None
    has_next = wn is not None
    in_specs = [
        pl.BlockSpec((n, _H), lambda i: (0, 0)),
        pl.BlockSpec((2, _H), lambda i: (0, 0)),
        pl.BlockSpec((bm, n), lambda i: (i, 0)),
    ]
    args = [hcat, a2, lap]
    if has_low:
        nlow = hcat_low.shape[0]
        in_specs += [
            pl.BlockSpec((nlow, bm), lambda i: (0, i)),
            pl.BlockSpec((nlow, _H), lambda i: (0, 1)),
        ]
        args += [bnd_low, hcat_low]
    if has_up:
        nup = hcat_up.shape[0]
        in_specs += [
            pl.BlockSpec((bm, nup), lambda i: (i, 0)),
            pl.BlockSpec((nup, _H), lambda i: (0, 2)),
        ]
        args += [bnd_up, hcat_up]
    if has_next:
        in_specs += [
            pl.BlockSpec((_H, _HC), lambda i: (0, 0)),
            pl.BlockSpec((1, _HC), lambda i: (0, 0)),
        ]
        args += [wn, bn]
    od = _HC if has_next else _H
    out_shape = [jax.ShapeDtypeStruct((n, od), jnp.float32)]
    out_specs = [pl.BlockSpec((bm, od), lambda i: (i, 0))]
    if emit_mask:
        out_shape.append(jax.ShapeDtypeStruct((n, n), jnp.int8))
        out_specs.append(pl.BlockSpec((bm, n), lambda i: (i, 0)))
    res = pl.pallas_call(
        functools.partial(_attn_body, has_low, has_up, has_next, emit_mask, bm),
        grid=(n // bm,),
        in_specs=in_specs,
        out_specs=out_specs,
        out_shape=out_shape,
        scratch_shapes=[pltpu.VMEM((1, n), jnp.float32)],
    )(*args)
    return res if emit_mask else (res[0], None)


def _head_body(s0, s1, s2, s3, x0, x1, x2, x3, w_ref, b_ref, o_ref):
    ps = jnp.dot(s0[...], x0[...], preferred_element_type=jnp.float32)
    ps = ps + jnp.dot(s1[...], x1[...], preferred_element_type=jnp.float32)
    ps = ps + jnp.dot(s2[...], x2[...], preferred_element_type=jnp.float32)
    ps = ps + jnp.dot(s3[...], x3[...], preferred_element_type=jnp.float32)
    feat = ps.reshape(1, 2 * _H)
    o_ref[...] = jnp.dot(feat, w_ref[...], preferred_element_type=jnp.float32) + b_ref[...]


def kernel(emb0, emb1, emb2, emb3, lap0, lap1, lap2, lap3, bnd1, bnd2, bnd3, params, order, idx, rel):
    embs = [emb0, emb1, emb2, emb3]
    laps = [lap0, lap1, lap2, lap3]
    bnds = [None, bnd1, bnd2, bnd3]
    lay = params["layers"]
    wcats = [jnp.concatenate([l["W"], l["W_low"], l["W_up"]], axis=1) for l in lay]
    bcats = [
        jnp.concatenate([l["b"], jnp.zeros((2 * _H,), jnp.float32)]).reshape(1, _HC)
        for l in lay
    ]
    a2s = [jnp.concatenate([l["a_src"].T, l["a_dst"].T], axis=0) for l in lay]
    b_lin2 = params["b_lin"].reshape(1, _H)

    hcats = [
        _lin_stage(embs[j], params["W_lin"], b_lin2, wcats[0], bcats[0]) for j in range(4)
    ]

    bms = [256, 256, 256, 256]
    masks = [None] * 4
    for i in range(4):
        wn, bn = (wcats[i + 1], bcats[i + 1]) if i < 3 else (None, None)
        new = []
        for j in range(4):
            hc, mk = _attn_stage(
                hcats[j], a2s[i],
                laps[j] if i == 0 else masks[j],
                bnds[j] if j > 0 else None,
                hcats[j - 1] if j > 0 else None,
                bnds[j + 1] if j < 3 else None,
                hcats[j + 1] if j < 3 else None,
                wn, bn, bms[j], emit_mask=(i == 0),
            )
            new.append(hc)
            if i == 0:
                masks[j] = mk
        hcats = new

    ss = []
    for j in range(4):
        n = _NS[j]
        sel = jnp.where(order == j, 1.0, 0.0)
        onehot = jnp.where(jnp.arange(n, dtype=jnp.int32) == idx, sel, 0.0)
        ss.append(jnp.stack([jnp.ones((n,), jnp.float32), onehot]))
    out = pl.pallas_call(
        _head_body,
        out_shape=jax.ShapeDtypeStruct((1, 2 * _H // 4), jnp.float32),
    )(ss[0], ss[1], ss[2], ss[3], hcats[0], hcats[1], hcats[2], hcats[3],
      params["W_rel"], params["b_rel"].reshape(1, -1))
    nz = jnp.nonzero(rel, size=out.shape[1])[0]
    return out[0][nz]


# speedup vs baseline: 1.1336x; 1.1336x over previous
import functools

import jax
import jax.numpy as jnp
from jax.experimental import pallas as pl
from jax.experimental.pallas import tpu as pltpu

_NS = [1024, 2048, 1536, 512]
_H = 256
_HC = 3 * _H


def _lin_body(x_ref, wl_ref, bl_ref, wc_ref, bc_ref, o_ref):
    x = jnp.dot(x_ref[...], wl_ref[...], preferred_element_type=jnp.float32)
    x = x + bl_ref[...]
    o_ref[...] = jnp.dot(x, wc_ref[...], preferred_element_type=jnp.float32) + bc_ref[...]


def _lin_stage(emb, w_lin, b_lin2, wc, bc, bm=512):
    n, c = emb.shape
    return pl.pallas_call(
        _lin_body,
        grid=(n // bm,),
        in_specs=[
            pl.BlockSpec((bm, c), lambda i: (i, 0)),
            pl.BlockSpec((c, _H), lambda i: (0, 0)),
            pl.BlockSpec((1, _H), lambda i: (0, 0)),
            pl.BlockSpec((_H, _HC), lambda i: (0, 0)),
            pl.BlockSpec((1, _HC), lambda i: (0, 0)),
        ],
        out_specs=pl.BlockSpec((bm, _HC), lambda i: (i, 0)),
        out_shape=jax.ShapeDtypeStruct((n, _HC), jnp.float32),
    )(emb, w_lin, b_lin2, wc, bc)


def _attn_body(has_low, has_up, has_next, emit_mask, bm, *refs):
    it = iter(refs)
    h_ref = next(it)
    a_ref = next(it)
    lap_ref = next(it)
    if has_low:
        bl_ref = next(it)
        ylow_ref = next(it)
    if has_up:
        bu_ref = next(it)
        yup_ref = next(it)
    if has_next:
        wn_ref = next(it)
        bn_ref = next(it)
    o_ref = next(it)
    if emit_mask:
        m_ref = next(it)

    i = pl.program_id(0)
    h = h_ref[...]
    hb = h_ref[pl.ds(i * bm, bm), :]
    a = a_ref[...]

    acc = None
    if has_low:
        acc = jax.lax.dot_general(
            bl_ref[...], ylow_ref[...],
            dimension_numbers=(((0,), (0,)), ((), ())),
            preferred_element_type=jnp.float32,
        )
    if has_up:
        up = jnp.dot(bu_ref[...], yup_ref[...], preferred_element_type=jnp.float32)
        acc = up if acc is None else acc + up

    s_dst = jnp.sum(h * a[1:2, :], axis=1)[None, :]
    s_src = jnp.sum(hb * a[0:1, :], axis=1, keepdims=True)
    e = s_src + s_dst
    e = jnp.maximum(e, 0.2 * e)
    nz = lap_ref[...] != 0
    if emit_mask:
        m_ref[...] = nz.astype(jnp.int8)
    e = jnp.where(nz, e, -1e9)
    m = jnp.max(e, axis=1, keepdims=True)
    p = jnp.exp(e - m)
    out = jnp.dot(p, h, preferred_element_type=jnp.float32)
    out = out / jnp.sum(p, axis=1, keepdims=True)
    if acc is not None:
        out = out + acc
    x = jnp.maximum(out, 0.0)
    if has_next:
        o_ref[...] = jnp.dot(x, wn_ref[...], preferred_element_type=jnp.float32) + bn_ref[...]
    else:
        o_ref[...] = x


def _attn_stage(hcat, a2, lap, bnd_low, hcat_low, bnd_up, hcat_up, wn, bn, bm, emit_mask):
    n = hcat.shape[0]
    has_low = bnd_low is not None
    has_up = bnd_up is not None
    has_next = wn is not None
    in_specs = [
        pl.BlockSpec((n, _H), lambda i: (0, 0)),
        pl.BlockSpec((2, _H), lambda i: (0, 0)),
        pl.BlockSpec((bm, n), lambda i: (i, 0)),
    ]
    args = [hcat, a2, lap]
    if has_low:
        nlow = hcat_low.shape[0]
        in_specs += [
            pl.BlockSpec((nlow, bm), lambda i: (0, i)),
            pl.BlockSpec((nlow, _H), lambda i: (0, 1)),
        ]
        args += [bnd_low, hcat_low]
    if has_up:
        nup = hcat_up.shape[0]
        in_specs += [
            pl.BlockSpec((bm, nup), lambda i: (i, 0)),
            pl.BlockSpec((nup, _H), lambda i: (0, 2)),
        ]
        args += [bnd_up, hcat_up]
    if has_next:
        in_specs += [
            pl.BlockSpec((_H, _HC), lambda i: (0, 0)),
            pl.BlockSpec((1, _HC), lambda i: (0, 0)),
        ]
        args += [wn, bn]
    od = _HC if has_next else _H
    out_shape = [jax.ShapeDtypeStruct((n, od), jnp.float32)]
    out_specs = [pl.BlockSpec((bm, od), lambda i: (i, 0))]
    if emit_mask:
        out_shape.append(jax.ShapeDtypeStruct((n, n), jnp.int8))
        out_specs.append(pl.BlockSpec((bm, n), lambda i: (i, 0)))
    res = pl.pallas_call(
        functools.partial(_attn_body, has_low, has_up, has_next, emit_mask, bm),
        grid=(n // bm,),
        in_specs=in_specs,
        out_specs=out_specs,
        out_shape=out_shape,
    )(*args)
    return res if emit_mask else (res[0], None)


def _head_body(s0, s1, s2, s3, x0, x1, x2, x3, w_ref, b_ref, o_ref):
    ps = jnp.dot(s0[...], x0[...], preferred_element_type=jnp.float32)
    ps = ps + jnp.dot(s1[...], x1[...], preferred_element_type=jnp.float32)
    ps = ps + jnp.dot(s2[...], x2[...], preferred_element_type=jnp.float32)
    ps = ps + jnp.dot(s3[...], x3[...], preferred_element_type=jnp.float32)
    feat = ps.reshape(1, 2 * _H)
    o_ref[...] = jnp.dot(feat, w_ref[...], preferred_element_type=jnp.float32) + b_ref[...]


def kernel(emb0, emb1, emb2, emb3, lap0, lap1, lap2, lap3, bnd1, bnd2, bnd3, params, order, idx, rel):
    embs = [emb0, emb1, emb2, emb3]
    laps = [lap0, lap1, lap2, lap3]
    bnds = [None, bnd1, bnd2, bnd3]
    lay = params["layers"]
    wcats = [jnp.concatenate([l["W"], l["W_low"], l["W_up"]], axis=1) for l in lay]
    bcats = [
        jnp.concatenate([l["b"], jnp.zeros((2 * _H,), jnp.float32)]).reshape(1, _HC)
        for l in lay
    ]
    a2s = [jnp.concatenate([l["a_src"].T, l["a_dst"].T], axis=0) for l in lay]
    b_lin2 = params["b_lin"].reshape(1, _H)

    hcats = [
        _lin_stage(embs[j], params["W_lin"], b_lin2, wcats[0], bcats[0]) for j in range(4)
    ]

    bms = [256, 256, 256, 256]
    masks = [None] * 4
    for i in range(4):
        wn, bn = (wcats[i + 1], bcats[i + 1]) if i < 3 else (None, None)
        new = []
        for j in range(4):
            hc, mk = _attn_stage(
                hcats[j], a2s[i],
                laps[j] if i == 0 else masks[j],
                bnds[j] if j > 0 else None,
                hcats[j - 1] if j > 0 else None,
                bnds[j + 1] if j < 3 else None,
                hcats[j + 1] if j < 3 else None,
                wn, bn, bms[j], emit_mask=(i == 0),
            )
            new.append(hc)
            if i == 0:
                masks[j] = mk
        hcats = new

    ss = []
    for j in range(4):
        n = _NS[j]
        sel = jnp.where(order == j, 1.0, 0.0)
        onehot = jnp.where(jnp.arange(n, dtype=jnp.int32) == idx, sel, 0.0)
        ss.append(jnp.stack([jnp.ones((n,), jnp.float32), onehot]))
    out = pl.pallas_call(
        _head_body,
        out_shape=jax.ShapeDtypeStruct((1, 2 * _H // 4), jnp.float32),
    )(ss[0], ss[1], ss[2], ss[3], hcats[0], hcats[1], hcats[2], hcats[3],
      params["W_rel"], params["b_rel"].reshape(1, -1))
    nz = jnp.nonzero(rel, size=out.shape[1])[0]
    return out[0][nz]


# speedup vs baseline: 1.2001x; 1.0587x over previous
import functools

import jax
import jax.numpy as jnp
from jax.experimental import pallas as pl
from jax.experimental.pallas import tpu as pltpu

_NS = [1024, 2048, 1536, 512]
_H = 256
_HC = 3 * _H


def _lin_body(x_ref, wl_ref, bl_ref, wc_ref, bc_ref, o_ref):
    x = jnp.dot(x_ref[...], wl_ref[...], preferred_element_type=jnp.float32)
    x = x + bl_ref[...]
    o_ref[...] = jnp.dot(x, wc_ref[...], preferred_element_type=jnp.float32) + bc_ref[...]


def _lin_stage(emb, w_lin, b_lin2, wc, bc, bm=512):
    n, c = emb.shape
    return pl.pallas_call(
        _lin_body,
        grid=(n // bm,),
        in_specs=[
            pl.BlockSpec((bm, c), lambda i: (i, 0)),
            pl.BlockSpec((c, _H), lambda i: (0, 0)),
            pl.BlockSpec((1, _H), lambda i: (0, 0)),
            pl.BlockSpec((_H, _HC), lambda i: (0, 0)),
            pl.BlockSpec((1, _HC), lambda i: (0, 0)),
        ],
        out_specs=pl.BlockSpec((bm, _HC), lambda i: (i, 0)),
        out_shape=jax.ShapeDtypeStruct((n, _HC), jnp.float32),
    )(emb, w_lin, b_lin2, wc, bc)


def _attn_body(has_low, has_up, has_next, emit_mask, bm, *refs):
    it = iter(refs)
    h_ref = next(it)
    a_ref = next(it)
    lap_ref = next(it)
    if has_low:
        bl_ref = next(it)
        ylow_ref = next(it)
    if has_up:
        bu_ref = next(it)
        yup_ref = next(it)
    if has_next:
        wn_ref = next(it)
        bn_ref = next(it)
    o_ref = next(it)
    if emit_mask:
        m_ref = next(it)

    i = pl.program_id(0)
    h = h_ref[...]
    hb = h_ref[pl.ds(i * bm, bm), :]
    a = a_ref[...]

    acc = None
    if has_low:
        acc = jax.lax.dot_general(
            bl_ref[...], ylow_ref[...],
            dimension_numbers=(((0,), (0,)), ((), ())),
            preferred_element_type=jnp.float32,
        )
    if has_up:
        up = jnp.dot(bu_ref[...], yup_ref[...], preferred_element_type=jnp.float32)
        acc = up if acc is None else acc + up

    s_dst = jnp.sum(h * a[1:2, :], axis=1)[None, :]
    s_src = jnp.sum(hb * a[0:1, :], axis=1, keepdims=True)
    e = s_src + s_dst
    e = jnp.maximum(e, 0.2 * e)
    nz = lap_ref[...] != 0
    if emit_mask:
        m_ref[...] = nz.astype(jnp.int8)
    e = jnp.where(nz, e, -1e9)
    m = jnp.max(e, axis=1, keepdims=True)
    p = jnp.exp(e - m)
    out = jnp.dot(p, h, preferred_element_type=jnp.float32)
    out = out / jnp.sum(p, axis=1, keepdims=True)
    if acc is not None:
        out = out + acc
    x = jnp.maximum(out, 0.0)
    if has_next:
        o_ref[...] = jnp.dot(x, wn_ref[...], preferred_element_type=jnp.float32) + bn_ref[...]
    else:
        o_ref[...] = x


def _attn_stage(hcat, a2, lap, bnd_low, hcat_low, bnd_up, hcat_up, wn, bn, bm, emit_mask):
    n = hcat.shape[0]
    has_low = bnd_low is not None
    has_up = bnd_up is not None
    has_next = wn is not None
    in_specs = [
        pl.BlockSpec((n, _H), lambda i: (0, 0)),
        pl.BlockSpec((2, _H), lambda i: (0, 0)),
        pl.BlockSpec((bm, n), lambda i: (i, 0)),
    ]
    args = [hcat, a2, lap]
    if has_low:
        nlow = hcat_low.shape[0]
        in_specs += [
            pl.BlockSpec((nlow, bm), lambda i: (0, i)),
            pl.BlockSpec((nlow, _H), lambda i: (0, 1)),
        ]
        args += [bnd_low, hcat_low]
    if has_up:
        nup = hcat_up.shape[0]
        in_specs += [
            pl.BlockSpec((bm, nup), lambda i: (i, 0)),
            pl.BlockSpec((nup, _H), lambda i: (0, 2)),
        ]
        args += [bnd_up, hcat_up]
    if has_next:
        in_specs += [
            pl.BlockSpec((_H, _HC), lambda i: (0, 0)),
            pl.BlockSpec((1, _HC), lambda i: (0, 0)),
        ]
        args += [wn, bn]
    od = _HC if has_next else _H
    out_shape = [jax.ShapeDtypeStruct((n, od), jnp.float32)]
    out_specs = [pl.BlockSpec((bm, od), lambda i: (i, 0))]
    if emit_mask:
        out_shape.append(jax.ShapeDtypeStruct((n, n), jnp.int8))
        out_specs.append(pl.BlockSpec((bm, n), lambda i: (i, 0)))
    res = pl.pallas_call(
        functools.partial(_attn_body, has_low, has_up, has_next, emit_mask, bm),
        grid=(n // bm,),
        in_specs=in_specs,
        out_specs=out_specs,
        out_shape=out_shape,
    )(*args)
    return res if emit_mask else (res[0], None)


def _head_body(s0, s1, s2, s3, x0, x1, x2, x3, w_ref, b_ref, o_ref):
    ps = jnp.dot(s0[...], x0[...], preferred_element_type=jnp.float32)
    ps = ps + jnp.dot(s1[...], x1[...], preferred_element_type=jnp.float32)
    ps = ps + jnp.dot(s2[...], x2[...], preferred_element_type=jnp.float32)
    ps = ps + jnp.dot(s3[...], x3[...], preferred_element_type=jnp.float32)
    feat = ps.reshape(1, 2 * _H)
    o_ref[...] = jnp.dot(feat, w_ref[...], preferred_element_type=jnp.float32) + b_ref[...]


def kernel(emb0, emb1, emb2, emb3, lap0, lap1, lap2, lap3, bnd1, bnd2, bnd3, params, order, idx, rel):
    embs = [emb0, emb1, emb2, emb3]
    laps = [lap0, lap1, lap2, lap3]
    bnds = [None, bnd1, bnd2, bnd3]
    lay = params["layers"]
    wcats = [jnp.concatenate([l["W"], l["W_low"], l["W_up"]], axis=1) for l in lay]
    bcats = [
        jnp.concatenate([l["b"], jnp.zeros((2 * _H,), jnp.float32)]).reshape(1, _HC)
        for l in lay
    ]
    a2s = [jnp.concatenate([l["a_src"].T, l["a_dst"].T], axis=0) for l in lay]
    b_lin2 = params["b_lin"].reshape(1, _H)

    hcats = [
        _lin_stage(embs[j], params["W_lin"], b_lin2, wcats[0], bcats[0]) for j in range(4)
    ]

    bms = [512, 512, 512, 512]
    masks = [None] * 4
    for i in range(4):
        wn, bn = (wcats[i + 1], bcats[i + 1]) if i < 3 else (None, None)
        new = []
        for j in range(4):
            hc, mk = _attn_stage(
                hcats[j], a2s[i],
                laps[j] if i == 0 else masks[j],
                bnds[j] if j > 0 else None,
                hcats[j - 1] if j > 0 else None,
                bnds[j + 1] if j < 3 else None,
                hcats[j + 1] if j < 3 else None,
                wn, bn, bms[j], emit_mask=(i == 0),
            )
            new.append(hc)
            if i == 0:
                masks[j] = mk
        hcats = new

    ss = []
    for j in range(4):
        n = _NS[j]
        sel = jnp.where(order == j, 1.0, 0.0)
        onehot = jnp.where(jnp.arange(n, dtype=jnp.int32) == idx, sel, 0.0)
        ss.append(jnp.stack([jnp.ones((n,), jnp.float32), onehot]))
    out = pl.pallas_call(
        _head_body,
        out_shape=jax.ShapeDtypeStruct((1, 2 * _H // 4), jnp.float32),
    )(ss[0], ss[1], ss[2], ss[3], hcats[0], hcats[1], hcats[2], hcats[3],
      params["W_rel"], params["b_rel"].reshape(1, -1))
    nz = jnp.nonzero(rel, size=out.shape[1])[0]
    return out[0][nz]
